# Initial kernel scaffold; baseline (speedup 1.0000x reference)
import jax
import jax.numpy as jnp
from jax.experimental import pallas as pl


def kernel(x, edge_index, edge_index_min, W1l, b1l, W1r, W2l, b2l, W2r):
    raise NotImplementedError("write your pallas kernel here")


# speedup vs baseline: 3.7166x; 3.7166x over previous
import functools

import jax
import jax.numpy as jnp
from jax import lax
from jax.experimental import pallas as pl
from jax.experimental.pallas import tpu as pltpu
from jax.experimental.pallas import tpu_sc as plsc

N = 10000
NP = 10240
D_IN = 256
D_HID = 512
D_OUT = 256
E_I = 160000
E_U = 80000
CB = 128
K = 128
NTILES = 16
RPT = NP // NTILES
EPC = 64
NTAIL = N % EPC


def _seg_mean_kernel_body(has_extra, args):
    if has_extra:
        (xflat, src_u, dst_u, src_i, dst_i, extra, out,
         acc, cstage, raw, flat, dstb, rows_a, rows_b, cnt, ctmp, inv, sem) = args
    else:
        (xflat, src_u, dst_u, src_i, dst_i, out,
         acc, cstage, raw, flat, dstb, rows_a, cnt, ctmp, inv, sem) = args
        rows_b = None

    cid = lax.axis_index("c")
    tid = lax.axis_index("s")
    r0 = tid * RPT
    zeros16 = jnp.zeros((16,), jnp.float32)
    ones16 = jnp.ones((16,), jnp.float32)

    for g, (srcr, dstr, e_total) in enumerate(
            ((src_u, dst_u, E_U), (src_i, dst_i, E_I))):
        chunks_total = e_total // K
        base = chunks_total // NTILES
        nc_t = jnp.where(tid == NTILES - 1,
                         chunks_total - (NTILES - 1) * base, base)
        lo_e = tid * (base * K)

        def _zrow(r, _):
            for c in range(CB // 16):
                rows_a[r, pl.ds(c * 16, 16)] = zeros16
            return 0
        lax.fori_loop(0, K, _zrow, 0)
        for z in range(RPT // K):
            pltpu.sync_copy(rows_a, acc.at[pl.ds(r0 + z * K, K), :])

        def _zcnt(i, _):
            cnt[pl.ds(i * 16, 16)] = zeros16
            return 0
        lax.fori_loop(0, NP // 16, _zcnt, 0)
        plsc.subcore_barrier()

        off = 2 * g * N + cid

        def _chunk(j, _):
            e0 = lo_e + j * K
            pltpu.sync_copy(srcr.at[pl.ds(e0, K)], raw)
            pltpu.sync_copy(dstr.at[pl.ds(e0, K)], dstb)
            for k2 in range(K // 16):
                sv = raw[pl.ds(k2 * 16, 16)]
                flat[pl.ds(k2 * 16, 16)] = sv + sv + off
                dv = dstb[pl.ds(k2 * 16, 16)]
                plsc.addupdate_scatter(cnt, [dv], ones16)
            pltpu.async_copy(xflat.at[flat], rows_a, sem).wait()
            pltpu.sync_copy(rows_a, acc.at[dstb], add=True)
            return 0
        lax.fori_loop(0, nc_t, _chunk, 0)

        pltpu.sync_copy(cnt, cstage.at[tid])
        plsc.subcore_barrier()

        def _zinv(i, _):
            inv[pl.ds(i * 16, 16)] = zeros16
            return 0
        lax.fori_loop(0, RPT // 16, _zinv, 0)
        for k in range(NTILES):
            pltpu.sync_copy(cstage.at[k, pl.ds(r0, RPT)], ctmp)

            def _radd(i, _):
                inv[pl.ds(i * 16, 16)] = (inv[pl.ds(i * 16, 16)]
                                          + ctmp[pl.ds(i * 16, 16)])
                return 0
            lax.fori_loop(0, RPT // 16, _radd, 0)

        def _rinv(i, _):
            v = inv[pl.ds(i * 16, 16)]
            inv[pl.ds(i * 16, 16)] = 1.0 / jnp.maximum(v, 1.0)
            return 0
        lax.fori_loop(0, RPT // 16, _rinv, 0)

        for ch in range(RPT // EPC):
            start = r0 + ch * EPC
            pltpu.sync_copy(acc.at[pl.ds(start, EPC), :],
                            rows_a.at[pl.ds(0, EPC)])
            if has_extra:
                @pl.when(start + EPC <= N)
                def _full_in():
                    pltpu.sync_copy(extra.at[g, pl.ds(start, EPC), cid], rows_b)

                @pl.when(jnp.logical_and(start < N, start + EPC > N))
                def _part_in():
                    pltpu.sync_copy(extra.at[g, pl.ds(start, NTAIL), cid],
                                    rows_b.at[pl.ds(0, NTAIL)])

            def _scale(r, _):
                vv = inv[pl.ds(ch * EPC + r, 16)]
                svec = jnp.full((16,), vv[0], jnp.float32)
                for c in range(CB // 16):
                    val = rows_a[r, pl.ds(c * 16, 16)] * svec
                    if has_extra:
                        val = val + rows_b[r, pl.ds(c * 16, 16)]
                    rows_a[r, pl.ds(c * 16, 16)] = val
                return 0
            lax.fori_loop(0, EPC, _scale, 0)

            @pl.when(start + EPC <= N)
            def _full_out():
                pltpu.sync_copy(rows_a.at[pl.ds(0, EPC)],
                                out.at[g, pl.ds(start, EPC), cid])

            @pl.when(jnp.logical_and(start < N, start + EPC > N))
            def _part_out():
                pltpu.sync_copy(rows_a.at[pl.ds(0, NTAIL)],
                                out.at[g, pl.ds(start, NTAIL), cid])

        plsc.subcore_barrier()


def _make_seg_mean(has_extra):
    mesh = plsc.VectorSubcoreMesh(core_axis_name="c", subcore_axis_name="s")
    scratch = [
        pltpu.VMEM_SHARED((NP, CB), jnp.float32),
        pltpu.VMEM_SHARED((NTILES, NP), jnp.float32),
        pltpu.VMEM((K,), jnp.int32),
        pltpu.VMEM((K,), jnp.int32),
        pltpu.VMEM((K,), jnp.int32),
        pltpu.VMEM((K, CB), jnp.float32),
    ]
    if has_extra:
        scratch.append(pltpu.VMEM((EPC, CB), jnp.float32))
    scratch += [
        pltpu.VMEM((NP,), jnp.float32),
        pltpu.VMEM((RPT,), jnp.float32),
        pltpu.VMEM((RPT + 16,), jnp.float32),
        pltpu.SemaphoreType.DMA,
    ]

    @functools.partial(
        pl.kernel,
        out_type=jax.ShapeDtypeStruct((2, N, 2, CB), jnp.float32),
        mesh=mesh,
        scratch_types=scratch,
        compiler_params=pltpu.CompilerParams(needs_layout_passes=False),
    )
    def k(*args):
        _seg_mean_kernel_body(has_extra, args)

    return k


_seg_mean_cache = {}


def _get_seg_mean(has_extra):
    if has_extra not in _seg_mean_cache:
        _seg_mean_cache[has_extra] = _make_seg_mean(has_extra)
    return _seg_mean_cache[has_extra]


BM = 1000


def _tc_body(mean_r, x_r, w1l_r, w1r_r, b1l_r, w2l_r, w2r_r, b2l_r,
             hl_r, hr_r):
    dn = (((1,), (1,)), ((), ()))
    m = mean_r[0]
    xx = x_r[0]
    h1 = lax.dot_general(m, w1l_r[...], dn, preferred_element_type=jnp.float32)
    h1 = h1 + lax.dot_general(xx, w1r_r[...], dn,
                              preferred_element_type=jnp.float32)
    h1 = jnp.maximum(h1 + b1l_r[...], 0.0)
    hl_r[0] = lax.dot_general(h1, w2l_r[...], dn,
                              preferred_element_type=jnp.float32)
    hr_r[0] = (lax.dot_general(h1, w2r_r[...], dn,
                               preferred_element_type=jnp.float32)
               + b2l_r[...])


def _tc_fused(mean1, x, W1l, b1l, W1r, W2l, b2l, W2r):
    grid = (2, N // BM)
    row_spec = pl.BlockSpec((1, BM, D_IN), lambda g, i: (g, i, 0))
    w_spec = pl.BlockSpec((D_HID, D_IN), lambda g, i: (0, 0))
    w2_spec = pl.BlockSpec((D_OUT, D_HID), lambda g, i: (0, 0))
    b1_spec = pl.BlockSpec((1, D_HID), lambda g, i: (0, 0))
    b2_spec = pl.BlockSpec((1, D_OUT), lambda g, i: (0, 0))
    out_spec = pl.BlockSpec((1, BM, D_OUT), lambda g, i: (g, i, 0))
    return pl.pallas_call(
        _tc_body,
        grid=grid,
        in_specs=[row_spec, row_spec, w_spec, w_spec, b1_spec,
                  w2_spec, w2_spec, b2_spec],
        out_specs=[out_spec, out_spec],
        out_shape=[jax.ShapeDtypeStruct((2, N, D_OUT), jnp.float32),
                   jax.ShapeDtypeStruct((2, N, D_OUT), jnp.float32)],
        compiler_params=pltpu.CompilerParams(
            dimension_semantics=("parallel", "parallel")),
    )(mean1, x, W1l, W1r, b1l.reshape(1, D_HID), W2l, W2r,
      b2l.reshape(1, D_OUT))


def kernel(x, edge_index, edge_index_min, W1l, b1l, W1r, W2l, b2l, W2r):
    src_i = edge_index[0].astype(jnp.int32)
    dst_i = edge_index[1].astype(jnp.int32)
    src_u = edge_index_min[0].astype(jnp.int32)
    dst_u = edge_index_min[1].astype(jnp.int32)

    xflat = x.reshape(2 * N * 2, CB)
    mean1 = _get_seg_mean(False)(xflat, src_u, dst_u, src_i, dst_i)
    mean1 = mean1.reshape(2, N, D_IN)

    hl, hr = _tc_fused(mean1, x, W1l, b1l, W1r, W2l, b2l, W2r)

    out = _get_seg_mean(True)(hl.reshape(2 * N * 2, CB), src_u, dst_u,
                              src_i, dst_i, hr.reshape(2, N, 2, CB))
    out = out.reshape(2, N, D_OUT)
    return out[0], out[1]


# speedup vs baseline: 5.8865x; 1.5838x over previous
import functools

import jax
import jax.numpy as jnp
from jax import lax
from jax.experimental import pallas as pl
from jax.experimental.pallas import tpu as pltpu
from jax.experimental.pallas import tpu_sc as plsc

N = 10000
NP = 10240
D_IN = 256
D_HID = 512
D_OUT = 256
E_I = 160000
E_U = 80000
CB = 128
K = 128
NTILES = 16
RPT = NP // NTILES
EPC = 128
NTAIL = N % EPC


def _seg_mean_kernel_body(compute_counts, args):
    if compute_counts:
        (xflat, src_u, dst_u, src_i, dst_i, out, inv_out, cstage,
         acc, raw_a, flat_a, dst_a, raw_b, flat_b, dst_b,
         rows_a, rows_b, cnt, ctmp, inv,
         is_a, is_b, gs_a, gs_b, ss_a, ss_b) = args
        extra = None
    else:
        (xflat, src_u, dst_u, src_i, dst_i, inv_in, extra, out,
         acc, raw_a, flat_a, dst_a, raw_b, flat_b, dst_b,
         rows_a, rows_b, inv,
         is_a, is_b, gs_a, gs_b, ss_a, ss_b) = args

    has_extra = extra is not None
    cid = lax.axis_index("c")
    tid = lax.axis_index("s")
    r0 = tid * RPT
    zeros16 = jnp.zeros((16,), jnp.float32)
    ones16 = jnp.ones((16,), jnp.float32)

    for g, (srcr, dstr, e_total) in enumerate(
            ((src_u, dst_u, E_U), (src_i, dst_i, E_I))):
        chunks_total = e_total // K
        base = chunks_total // NTILES
        nc = jnp.where(tid == NTILES - 1,
                       chunks_total - (NTILES - 1) * base, base)
        lo_c = tid * base

        off = 2 * g * N + cid

        def idx_start(c, rb, db, sem):
            e0 = (lo_c + c) * K
            pltpu.async_copy(srcr.at[pl.ds(e0, K)], rb, sem)
            pltpu.async_copy(dstr.at[pl.ds(e0, K)], db, sem)

        def idx_wait(c, rb, db, sem):
            e0 = (lo_c + c) * K
            pltpu.make_async_copy(srcr.at[pl.ds(e0, K)], rb, sem).wait()
            pltpu.make_async_copy(dstr.at[pl.ds(e0, K)], db, sem).wait()

        def transform(rb, fb, db):
            for k2 in range(K // 16):
                sv = rb[pl.ds(k2 * 16, 16)]
                fb[pl.ds(k2 * 16, 16)] = sv + sv + off
                if compute_counts:
                    dv = db[pl.ds(k2 * 16, 16)]
                    plsc.addupdate_scatter(cnt, [dv], ones16)

        def gather_start(fb, rows, sem):
            pltpu.async_copy(xflat.at[fb], rows, sem)

        def gather_wait(fb, rows, sem):
            pltpu.make_async_copy(xflat.at[fb], rows, sem).wait()

        def scat_start(rows, db, sem):
            pltpu.async_copy(rows, acc.at[db], sem, add=True)

        def scat_wait(rows, db, sem):
            pltpu.make_async_copy(rows, acc.at[db], sem).wait()

        def _zrow(r, _):
            for c in range(CB // 16):
                rows_a[r, pl.ds(c * 16, 16)] = zeros16
            return 0
        lax.fori_loop(0, K, _zrow, 0)
        for z in range(RPT // K):
            pltpu.sync_copy(rows_a, acc.at[pl.ds(r0 + z * K, K), :])
        if compute_counts:
            def _zcnt(i, _):
                cnt[pl.ds(i * 16, 16)] = zeros16
                return 0
            lax.fori_loop(0, NP // 16, _zcnt, 0)
        plsc.subcore_barrier()

        idx_start(0, raw_a, dst_a, is_a)

        @pl.when(nc > 1)
        def _pro_b():
            idx_start(1, raw_b, dst_b, is_b)

        idx_wait(0, raw_a, dst_a, is_a)
        transform(raw_a, flat_a, dst_a)
        gather_start(flat_a, rows_a, gs_a)

        def _pair(p, _):
            c1 = 2 * p + 1
            c2 = c1 + 1
            c3 = c1 + 2
            has1 = c1 < nc

            @pl.when(has1)
            def _prep_b():
                idx_wait(c1, raw_b, dst_b, is_b)
                transform(raw_b, flat_b, dst_b)

            gather_wait(flat_a, rows_a, gs_a)
            scat_start(rows_a, dst_a, ss_a)

            @pl.when(has1)
            def _g_b():
                gather_start(flat_b, rows_b, gs_b)

            scat_wait(rows_a, dst_a, ss_a)

            @pl.when(c2 < nc)
            def _next_a():
                idx_start(c2, raw_a, dst_a, is_a)
                idx_wait(c2, raw_a, dst_a, is_a)
                transform(raw_a, flat_a, dst_a)
                gather_start(flat_a, rows_a, gs_a)

            @pl.when(has1)
            def _fin_b():
                gather_wait(flat_b, rows_b, gs_b)
                scat_start(rows_b, dst_b, ss_b)
                scat_wait(rows_b, dst_b, ss_b)

            @pl.when(c3 < nc)
            def _idx_b():
                idx_start(c3, raw_b, dst_b, is_b)
            return 0
        lax.fori_loop(0, (nc + 1) // 2, _pair, 0)

        if compute_counts:
            pltpu.sync_copy(cnt, cstage.at[g, tid])
            plsc.subcore_barrier()

            def _zinv(i, _):
                inv[pl.ds(i * 16, 16)] = zeros16
                return 0
            lax.fori_loop(0, RPT // 16, _zinv, 0)
            for k in range(NTILES):
                pltpu.sync_copy(cstage.at[g, k, pl.ds(r0, RPT)], ctmp)

                def _radd(i, _):
                    inv[pl.ds(i * 16, 16)] = (inv[pl.ds(i * 16, 16)]
                                              + ctmp[pl.ds(i * 16, 16)])
                    return 0
                lax.fori_loop(0, RPT // 16, _radd, 0)

            def _rinv(i, _):
                v = inv[pl.ds(i * 16, 16)]
                inv[pl.ds(i * 16, 16)] = 1.0 / jnp.maximum(v, 1.0)
                return 0
            lax.fori_loop(0, RPT // 16, _rinv, 0)

            @pl.when(cid == 0)
            def _winv():
                pltpu.sync_copy(inv.at[pl.ds(0, RPT)],
                                inv_out.at[g, pl.ds(r0, RPT)])
        else:
            pltpu.sync_copy(inv_in.at[g, pl.ds(r0, RPT)],
                            inv.at[pl.ds(0, RPT)])
            plsc.subcore_barrier()

        for ch in range(RPT // EPC):
            start = r0 + ch * EPC
            pltpu.sync_copy(acc.at[pl.ds(start, EPC), :], rows_a)
            if has_extra:
                @pl.when(start + EPC <= N)
                def _full_in():
                    pltpu.sync_copy(extra.at[g, pl.ds(start, EPC), cid],
                                    rows_b)

                @pl.when(jnp.logical_and(start < N, start + EPC > N))
                def _part_in():
                    pltpu.sync_copy(extra.at[g, pl.ds(start, NTAIL), cid],
                                    rows_b.at[pl.ds(0, NTAIL)])

            def _scale(r, _):
                vv = inv[pl.ds(ch * EPC + r, 16)]
                svec = jnp.full((16,), vv[0], jnp.float32)
                for c in range(CB // 16):
                    val = rows_a[r, pl.ds(c * 16, 16)] * svec
                    if has_extra:
                        val = val + rows_b[r, pl.ds(c * 16, 16)]
                    rows_a[r, pl.ds(c * 16, 16)] = val
                return 0
            lax.fori_loop(0, EPC, _scale, 0)

            @pl.when(start + EPC <= N)
            def _full_out():
                pltpu.sync_copy(rows_a, out.at[g, pl.ds(start, EPC), cid])

            @pl.when(jnp.logical_and(start < N, start + EPC > N))
            def _part_out():
                pltpu.sync_copy(rows_a.at[pl.ds(0, NTAIL)],
                                out.at[g, pl.ds(start, NTAIL), cid])


def _make_seg_mean(compute_counts):
    mesh = plsc.VectorSubcoreMesh(core_axis_name="c", subcore_axis_name="s")
    scratch = [
        pltpu.VMEM_SHARED((NP, CB), jnp.float32),
        pltpu.VMEM((K,), jnp.int32),
        pltpu.VMEM((K,), jnp.int32),
        pltpu.VMEM((K,), jnp.int32),
        pltpu.VMEM((K,), jnp.int32),
        pltpu.VMEM((K,), jnp.int32),
        pltpu.VMEM((K,), jnp.int32),
        pltpu.VMEM((K, CB), jnp.float32),
        pltpu.VMEM((K, CB), jnp.float32),
    ]
    if compute_counts:
        scratch += [
            pltpu.VMEM((NP,), jnp.float32),
            pltpu.VMEM((RPT,), jnp.float32),
        ]
    scratch += [
        pltpu.VMEM((RPT + 16,), jnp.float32),
        pltpu.SemaphoreType.DMA,
        pltpu.SemaphoreType.DMA,
        pltpu.SemaphoreType.DMA,
        pltpu.SemaphoreType.DMA,
        pltpu.SemaphoreType.DMA,
        pltpu.SemaphoreType.DMA,
    ]
    if compute_counts:
        out_type = [
            jax.ShapeDtypeStruct((2, N, 2, CB), jnp.float32),
            jax.ShapeDtypeStruct((2, NP), jnp.float32),
            jax.ShapeDtypeStruct((2, NTILES, NP), jnp.float32),
        ]
    else:
        out_type = jax.ShapeDtypeStruct((2, N, 2, CB), jnp.float32)

    @functools.partial(
        pl.kernel,
        out_type=out_type,
        mesh=mesh,
        scratch_types=scratch,
        compiler_params=pltpu.CompilerParams(needs_layout_passes=False),
    )
    def k(*args):
        _seg_mean_kernel_body(compute_counts, args)

    return k


_seg_mean_cache = {}


def _get_seg_mean(compute_counts):
    if compute_counts not in _seg_mean_cache:
        _seg_mean_cache[compute_counts] = _make_seg_mean(compute_counts)
    return _seg_mean_cache[compute_counts]


BM = 1000


def _tc_body(mean_r, x_r, w1l_r, w1r_r, b1l_r, w2l_r, w2r_r, b2l_r,
             hl_r, hr_r):
    dn = (((1,), (1,)), ((), ()))
    m = mean_r[0]
    xx = x_r[0]
    h1 = lax.dot_general(m, w1l_r[...], dn, preferred_element_type=jnp.float32)
    h1 = h1 + lax.dot_general(xx, w1r_r[...], dn,
                              preferred_element_type=jnp.float32)
    h1 = jnp.maximum(h1 + b1l_r[...], 0.0)
    hl_r[0] = lax.dot_general(h1, w2l_r[...], dn,
                              preferred_element_type=jnp.float32)
    hr_r[0] = (lax.dot_general(h1, w2r_r[...], dn,
                               preferred_element_type=jnp.float32)
               + b2l_r[...])


def _tc_fused(mean1, x, W1l, b1l, W1r, W2l, b2l, W2r):
    grid = (2, N // BM)
    row_spec = pl.BlockSpec((1, BM, D_IN), lambda g, i: (g, i, 0))
    w_spec = pl.BlockSpec((D_HID, D_IN), lambda g, i: (0, 0))
    w2_spec = pl.BlockSpec((D_OUT, D_HID), lambda g, i: (0, 0))
    b1_spec = pl.BlockSpec((1, D_HID), lambda g, i: (0, 0))
    b2_spec = pl.BlockSpec((1, D_OUT), lambda g, i: (0, 0))
    out_spec = pl.BlockSpec((1, BM, D_OUT), lambda g, i: (g, i, 0))
    return pl.pallas_call(
        _tc_body,
        grid=grid,
        in_specs=[row_spec, row_spec, w_spec, w_spec, b1_spec,
                  w2_spec, w2_spec, b2_spec],
        out_specs=[out_spec, out_spec],
        out_shape=[jax.ShapeDtypeStruct((2, N, D_OUT), jnp.float32),
                   jax.ShapeDtypeStruct((2, N, D_OUT), jnp.float32)],
        compiler_params=pltpu.CompilerParams(
            dimension_semantics=("parallel", "parallel")),
    )(mean1, x, W1l, W1r, b1l.reshape(1, D_HID), W2l, W2r,
      b2l.reshape(1, D_OUT))


def kernel(x, edge_index, edge_index_min, W1l, b1l, W1r, W2l, b2l, W2r):
    src_i = edge_index[0].astype(jnp.int32)
    dst_i = edge_index[1].astype(jnp.int32)
    src_u = edge_index_min[0].astype(jnp.int32)
    dst_u = edge_index_min[1].astype(jnp.int32)

    xflat = x.reshape(2 * N * 2, CB)
    mean1, invd, _ = _get_seg_mean(True)(xflat, src_u, dst_u, src_i, dst_i)
    mean1 = mean1.reshape(2, N, D_IN)

    hl, hr = _tc_fused(mean1, x, W1l, b1l, W1r, W2l, b2l, W2r)

    out = _get_seg_mean(False)(hl.reshape(2 * N * 2, CB), src_u, dst_u,
                               src_i, dst_i, invd, hr.reshape(2, N, 2, CB))
    out = out.reshape(2, N, D_OUT)
    return out[0], out[1]


# speedup vs baseline: 5.8938x; 1.0012x over previous
import functools

import jax
import jax.numpy as jnp
from jax import lax
from jax.experimental import pallas as pl
from jax.experimental.pallas import tpu as pltpu
from jax.experimental.pallas import tpu_sc as plsc

N = 10000
NP = 10240
D_IN = 256
D_HID = 512
D_OUT = 256
E_I = 160000
E_U = 80000
CB = 128
K = 128
NTILES = 16
RPT = NP // NTILES
EPC = 128
NTAIL = N % EPC


def _seg_mean_kernel_body(compute_counts, args):
    if compute_counts:
        (xflat, src_u, dst_u, src_i, dst_i, out, inv_out, cstage,
         acc, raw_a, flat_a, dst_a, raw_b, flat_b, dst_b,
         rows_a, rows_b, cnt, ctmp, inv,
         is_a, is_b, gs_a, gs_b, ss_a, ss_b) = args
        extra = None
    else:
        (xflat, src_u, dst_u, src_i, dst_i, inv_in, extra, out,
         acc, raw_a, flat_a, dst_a, raw_b, flat_b, dst_b,
         rows_a, rows_b, inv,
         is_a, is_b, gs_a, gs_b, ss_a, ss_b) = args

    has_extra = extra is not None
    cid = lax.axis_index("c")
    tid = lax.axis_index("s")
    r0 = tid * RPT
    zeros16 = jnp.zeros((16,), jnp.float32)
    ones16 = jnp.ones((16,), jnp.float32)

    for g, (srcr, dstr, e_total) in enumerate(
            ((src_u, dst_u, E_U), (src_i, dst_i, E_I))):
        chunks_total = e_total // K
        base = chunks_total // NTILES
        nc = jnp.where(tid == NTILES - 1,
                       chunks_total - (NTILES - 1) * base, base)
        lo_c = tid * base

        off = 2 * g * N + cid

        def idx_start(c, rb, db, sem):
            e0 = (lo_c + c) * K
            pltpu.async_copy(srcr.at[pl.ds(e0, K)], rb, sem)
            pltpu.async_copy(dstr.at[pl.ds(e0, K)], db, sem)

        def idx_wait(c, rb, db, sem):
            e0 = (lo_c + c) * K
            pltpu.make_async_copy(srcr.at[pl.ds(e0, K)], rb, sem).wait()
            pltpu.make_async_copy(dstr.at[pl.ds(e0, K)], db, sem).wait()

        def transform(rb, fb, db):
            for k2 in range(K // 16):
                sv = rb[pl.ds(k2 * 16, 16)]
                fb[pl.ds(k2 * 16, 16)] = sv + sv + off
                if compute_counts:
                    dv = db[pl.ds(k2 * 16, 16)]
                    plsc.addupdate_scatter(cnt, [dv], ones16)

        def gather_start(fb, rows, sem):
            pltpu.async_copy(xflat.at[fb], rows, sem)

        def gather_wait(fb, rows, sem):
            pltpu.make_async_copy(xflat.at[fb], rows, sem).wait()

        def scat_start(rows, db, sem):
            pltpu.async_copy(rows, acc.at[db], sem, add=True)

        def scat_wait(rows, db, sem):
            pltpu.make_async_copy(rows, acc.at[db], sem).wait()

        def _zrow(r, _):
            for c in range(CB // 16):
                rows_a[r, pl.ds(c * 16, 16)] = zeros16
            return 0
        lax.fori_loop(0, K, _zrow, 0)
        for z in range(RPT // K):
            pltpu.sync_copy(rows_a, acc.at[pl.ds(r0 + z * K, K), :])
        if compute_counts:
            def _zcnt(i, _):
                cnt[pl.ds(i * 16, 16)] = zeros16
                return 0
            lax.fori_loop(0, NP // 16, _zcnt, 0)
        plsc.subcore_barrier()

        idx_start(0, raw_a, dst_a, is_a)

        @pl.when(nc > 1)
        def _pro_b():
            idx_start(1, raw_b, dst_b, is_b)

        idx_wait(0, raw_a, dst_a, is_a)
        transform(raw_a, flat_a, dst_a)
        gather_start(flat_a, rows_a, gs_a)

        def _pair(p, _):
            c1 = 2 * p + 1
            c2 = c1 + 1
            c3 = c1 + 2
            has1 = c1 < nc

            @pl.when(has1)
            def _prep_b():
                idx_wait(c1, raw_b, dst_b, is_b)
                transform(raw_b, flat_b, dst_b)

            gather_wait(flat_a, rows_a, gs_a)
            scat_start(rows_a, dst_a, ss_a)

            @pl.when(has1)
            def _g_b():
                gather_start(flat_b, rows_b, gs_b)

            scat_wait(rows_a, dst_a, ss_a)

            @pl.when(c2 < nc)
            def _next_a():
                idx_start(c2, raw_a, dst_a, is_a)
                idx_wait(c2, raw_a, dst_a, is_a)
                transform(raw_a, flat_a, dst_a)
                gather_start(flat_a, rows_a, gs_a)

            @pl.when(has1)
            def _fin_b():
                gather_wait(flat_b, rows_b, gs_b)
                scat_start(rows_b, dst_b, ss_b)
                scat_wait(rows_b, dst_b, ss_b)

            @pl.when(c3 < nc)
            def _idx_b():
                idx_start(c3, raw_b, dst_b, is_b)
            return 0
        lax.fori_loop(0, (nc + 1) // 2, _pair, 0)

        if compute_counts:
            pltpu.sync_copy(cnt, cstage.at[g, tid])
            plsc.subcore_barrier()

            def _zinv(i, _):
                inv[pl.ds(i * 16, 16)] = zeros16
                return 0
            lax.fori_loop(0, RPT // 16, _zinv, 0)
            for k in range(NTILES):
                pltpu.sync_copy(cstage.at[g, k, pl.ds(r0, RPT)], ctmp)

                def _radd(i, _):
                    inv[pl.ds(i * 16, 16)] = (inv[pl.ds(i * 16, 16)]
                                              + ctmp[pl.ds(i * 16, 16)])
                    return 0
                lax.fori_loop(0, RPT // 16, _radd, 0)

            def _rinv(i, _):
                v = inv[pl.ds(i * 16, 16)]
                inv[pl.ds(i * 16, 16)] = 1.0 / jnp.maximum(v, 1.0)
                return 0
            lax.fori_loop(0, RPT // 16, _rinv, 0)

            @pl.when(cid == 0)
            def _winv():
                pltpu.sync_copy(inv.at[pl.ds(0, RPT)],
                                inv_out.at[g, pl.ds(r0, RPT)])
        else:
            pltpu.sync_copy(inv_in.at[g, pl.ds(r0, RPT)],
                            inv.at[pl.ds(0, RPT)])
            plsc.subcore_barrier()

        for ch in range(RPT // EPC):
            start = r0 + ch * EPC
            pltpu.sync_copy(acc.at[pl.ds(start, EPC), :], rows_a)
            if has_extra:
                @pl.when(start + EPC <= N)
                def _full_in():
                    pltpu.sync_copy(extra.at[g, pl.ds(start, EPC), cid],
                                    rows_b)

                @pl.when(jnp.logical_and(start < N, start + EPC > N))
                def _part_in():
                    pltpu.sync_copy(extra.at[g, pl.ds(start, NTAIL), cid],
                                    rows_b.at[pl.ds(0, NTAIL)])

            def _scale(r, _):
                vv = inv[pl.ds(ch * EPC + r, 16)]
                svec = jnp.full((16,), vv[0], jnp.float32)
                for c in range(CB // 16):
                    val = rows_a[r, pl.ds(c * 16, 16)] * svec
                    if has_extra:
                        val = val + rows_b[r, pl.ds(c * 16, 16)]
                    rows_a[r, pl.ds(c * 16, 16)] = val
                return 0
            lax.fori_loop(0, EPC, _scale, 0)

            @pl.when(start + EPC <= N)
            def _full_out():
                pltpu.sync_copy(rows_a, out.at[g, pl.ds(start, EPC), cid])

            @pl.when(jnp.logical_and(start < N, start + EPC > N))
            def _part_out():
                pltpu.sync_copy(rows_a.at[pl.ds(0, NTAIL)],
                                out.at[g, pl.ds(start, NTAIL), cid])


def _make_seg_mean(compute_counts):
    mesh = plsc.VectorSubcoreMesh(core_axis_name="c", subcore_axis_name="s")
    scratch = [
        pltpu.VMEM_SHARED((NP, CB), jnp.float32),
        pltpu.VMEM((K,), jnp.int32),
        pltpu.VMEM((K,), jnp.int32),
        pltpu.VMEM((K,), jnp.int32),
        pltpu.VMEM((K,), jnp.int32),
        pltpu.VMEM((K,), jnp.int32),
        pltpu.VMEM((K,), jnp.int32),
        pltpu.VMEM((K, CB), jnp.float32),
        pltpu.VMEM((K, CB), jnp.float32),
    ]
    if compute_counts:
        scratch += [
            pltpu.VMEM((NP,), jnp.float32),
            pltpu.VMEM((RPT,), jnp.float32),
        ]
    scratch += [
        pltpu.VMEM((RPT + 16,), jnp.float32),
        pltpu.SemaphoreType.DMA,
        pltpu.SemaphoreType.DMA,
        pltpu.SemaphoreType.DMA,
        pltpu.SemaphoreType.DMA,
        pltpu.SemaphoreType.DMA,
        pltpu.SemaphoreType.DMA,
    ]
    if compute_counts:
        out_type = [
            jax.ShapeDtypeStruct((2, N, 2, CB), jnp.float32),
            jax.ShapeDtypeStruct((2, NP), jnp.float32),
            jax.ShapeDtypeStruct((2, NTILES, NP), jnp.float32),
        ]
    else:
        out_type = jax.ShapeDtypeStruct((2, N, 2, CB), jnp.float32)

    @functools.partial(
        pl.kernel,
        out_type=out_type,
        mesh=mesh,
        scratch_types=scratch,
        compiler_params=pltpu.CompilerParams(needs_layout_passes=False),
    )
    def k(*args):
        _seg_mean_kernel_body(compute_counts, args)

    return k


_seg_mean_cache = {}


def _get_seg_mean(compute_counts):
    if compute_counts not in _seg_mean_cache:
        _seg_mean_cache[compute_counts] = _make_seg_mean(compute_counts)
    return _seg_mean_cache[compute_counts]


BM = 1000


def _tc_body(mean_r, x_r, w1l_r, w1r_r, b1l_r, w2l_r, w2r_r, b2l_r,
             hl_r, hr_r):
    dn = (((1,), (1,)), ((), ()))
    bf = jnp.bfloat16
    m = mean_r[0].astype(bf)
    xx = x_r[0].astype(bf)
    h1 = lax.dot_general(m, w1l_r[...].astype(bf), dn,
                         preferred_element_type=jnp.float32)
    h1 = h1 + lax.dot_general(xx, w1r_r[...].astype(bf), dn,
                              preferred_element_type=jnp.float32)
    h1 = jnp.maximum(h1 + b1l_r[...], 0.0).astype(bf)
    hl_r[0] = lax.dot_general(h1, w2l_r[...].astype(bf), dn,
                              preferred_element_type=jnp.float32)
    hr_r[0] = (lax.dot_general(h1, w2r_r[...].astype(bf), dn,
                               preferred_element_type=jnp.float32)
               + b2l_r[...])


def _tc_fused(mean1, x, W1l, b1l, W1r, W2l, b2l, W2r):
    grid = (2, N // BM)
    row_spec = pl.BlockSpec((1, BM, D_IN), lambda g, i: (g, i, 0))
    w_spec = pl.BlockSpec((D_HID, D_IN), lambda g, i: (0, 0))
    w2_spec = pl.BlockSpec((D_OUT, D_HID), lambda g, i: (0, 0))
    b1_spec = pl.BlockSpec((1, D_HID), lambda g, i: (0, 0))
    b2_spec = pl.BlockSpec((1, D_OUT), lambda g, i: (0, 0))
    out_spec = pl.BlockSpec((1, BM, D_OUT), lambda g, i: (g, i, 0))
    return pl.pallas_call(
        _tc_body,
        grid=grid,
        in_specs=[row_spec, row_spec, w_spec, w_spec, b1_spec,
                  w2_spec, w2_spec, b2_spec],
        out_specs=[out_spec, out_spec],
        out_shape=[jax.ShapeDtypeStruct((2, N, D_OUT), jnp.float32),
                   jax.ShapeDtypeStruct((2, N, D_OUT), jnp.float32)],
        compiler_params=pltpu.CompilerParams(
            dimension_semantics=("parallel", "parallel")),
    )(mean1, x, W1l, W1r, b1l.reshape(1, D_HID), W2l, W2r,
      b2l.reshape(1, D_OUT))


def kernel(x, edge_index, edge_index_min, W1l, b1l, W1r, W2l, b2l, W2r):
    src_i = edge_index[0].astype(jnp.int32)
    dst_i = edge_index[1].astype(jnp.int32)
    src_u = edge_index_min[0].astype(jnp.int32)
    dst_u = edge_index_min[1].astype(jnp.int32)

    xflat = x.reshape(2 * N * 2, CB)
    mean1, invd, _ = _get_seg_mean(True)(xflat, src_u, dst_u, src_i, dst_i)
    mean1 = mean1.reshape(2, N, D_IN)

    hl, hr = _tc_fused(mean1, x, W1l, b1l, W1r, W2l, b2l, W2r)

    out = _get_seg_mean(False)(hl.reshape(2 * N * 2, CB), src_u, dst_u,
                               src_i, dst_i, invd, hr.reshape(2, N, 2, CB))
    out = out.reshape(2, N, D_OUT)
    return out[0], out[1]


# speedup vs baseline: 7.0863x; 1.2023x over previous
import functools

import jax
import jax.numpy as jnp
from jax import lax
from jax.experimental import pallas as pl
from jax.experimental.pallas import tpu as pltpu
from jax.experimental.pallas import tpu_sc as plsc

N = 10000
NP = 10240
D_IN = 256
D_HID = 512
D_OUT = 256
E_I = 160000
E_U = 80000
CB = 128
K = 128
NTILES = 16
RPT = NP // NTILES
EPC = 128
NTAIL = N % EPC


def _seg_mean_kernel_body(compute_counts, args):
    if compute_counts:
        (xflat, src_u, dst_u, src_i, dst_i, out, inv_out, cstage,
         acc, raw_a, flat_a, dst_a, raw_b, flat_b, dst_b,
         rows_a, rows_b, cnt, ctmp, inv,
         is_a, is_b, gs_a, gs_b, ss_a, ss_b) = args
        extra = None
        outs = (out, out)
    else:
        (xflat, src_u, dst_u, src_i, dst_i, inv_in, extra, out_u, out_i,
         acc, raw_a, flat_a, dst_a, raw_b, flat_b, dst_b,
         rows_a, rows_b, inv,
         is_a, is_b, gs_a, gs_b, ss_a, ss_b) = args
        outs = (out_u, out_i)

    has_extra = extra is not None
    cid = lax.axis_index("c")
    tid = lax.axis_index("s")
    r0 = tid * RPT
    zeros16 = jnp.zeros((16,), jnp.float32)
    ones16 = jnp.ones((16,), jnp.float32)

    for g, (srcr, dstr, e_total) in enumerate(
            ((src_u, dst_u, E_U), (src_i, dst_i, E_I))):
        chunks_total = e_total // K
        base = chunks_total // NTILES
        nc = jnp.where(tid == NTILES - 1,
                       chunks_total - (NTILES - 1) * base, base)
        lo_c = tid * base

        off = g * N
        col0 = cid * CB

        def idx_start(c, rb, db, sem):
            e0 = (lo_c + c) * K
            pltpu.async_copy(srcr.at[pl.ds(e0, K)], rb, sem)
            pltpu.async_copy(dstr.at[pl.ds(e0, K)], db, sem)

        def idx_wait(c, rb, db, sem):
            e0 = (lo_c + c) * K
            pltpu.make_async_copy(srcr.at[pl.ds(e0, K)], rb, sem).wait()
            pltpu.make_async_copy(dstr.at[pl.ds(e0, K)], db, sem).wait()

        def transform(rb, fb, db):
            for k2 in range(K // 16):
                sv = rb[pl.ds(k2 * 16, 16)]
                fb[pl.ds(k2 * 16, 16)] = sv + off
                if compute_counts:
                    dv = db[pl.ds(k2 * 16, 16)]
                    plsc.addupdate_scatter(cnt, [dv], ones16)

        def gather_start(fb, rows, sem):
            pltpu.async_copy(xflat.at[fb, pl.ds(col0, CB)], rows, sem)

        def gather_wait(fb, rows, sem):
            pltpu.make_async_copy(xflat.at[fb, pl.ds(col0, CB)],
                                  rows, sem).wait()

        def scat_start(rows, db, sem):
            pltpu.async_copy(rows, acc.at[db], sem, add=True)

        def scat_wait(rows, db, sem):
            pltpu.make_async_copy(rows, acc.at[db], sem).wait()

        def _zrow(r, _):
            for c in range(CB // 16):
                rows_a[r, pl.ds(c * 16, 16)] = zeros16
            return 0
        lax.fori_loop(0, K, _zrow, 0)
        for z in range(RPT // K):
            pltpu.sync_copy(rows_a, acc.at[pl.ds(r0 + z * K, K), :])
        if compute_counts:
            def _zcnt(i, _):
                cnt[pl.ds(i * 16, 16)] = zeros16
                return 0
            lax.fori_loop(0, NP // 16, _zcnt, 0)
        plsc.subcore_barrier()

        idx_start(0, raw_a, dst_a, is_a)

        @pl.when(nc > 1)
        def _pro_b():
            idx_start(1, raw_b, dst_b, is_b)

        idx_wait(0, raw_a, dst_a, is_a)
        transform(raw_a, flat_a, dst_a)
        gather_start(flat_a, rows_a, gs_a)

        def _pair(p, _):
            c1 = 2 * p + 1
            c2 = c1 + 1
            c3 = c1 + 2
            has1 = c1 < nc

            @pl.when(has1)
            def _prep_b():
                idx_wait(c1, raw_b, dst_b, is_b)
                transform(raw_b, flat_b, dst_b)

            gather_wait(flat_a, rows_a, gs_a)
            scat_start(rows_a, dst_a, ss_a)

            @pl.when(has1)
            def _g_b():
                gather_start(flat_b, rows_b, gs_b)

            scat_wait(rows_a, dst_a, ss_a)

            @pl.when(c2 < nc)
            def _next_a():
                idx_start(c2, raw_a, dst_a, is_a)
                idx_wait(c2, raw_a, dst_a, is_a)
                transform(raw_a, flat_a, dst_a)
                gather_start(flat_a, rows_a, gs_a)

            @pl.when(has1)
            def _fin_b():
                gather_wait(flat_b, rows_b, gs_b)
                scat_start(rows_b, dst_b, ss_b)
                scat_wait(rows_b, dst_b, ss_b)

            @pl.when(c3 < nc)
            def _idx_b():
                idx_start(c3, raw_b, dst_b, is_b)
            return 0
        lax.fori_loop(0, (nc + 1) // 2, _pair, 0)

        if compute_counts:
            pltpu.sync_copy(cnt, cstage.at[g, tid])
            plsc.subcore_barrier()

            def _zinv(i, _):
                inv[pl.ds(i * 16, 16)] = zeros16
                return 0
            lax.fori_loop(0, RPT // 16, _zinv, 0)
            for k in range(NTILES):
                pltpu.sync_copy(cstage.at[g, k, pl.ds(r0, RPT)], ctmp)

                def _radd(i, _):
                    inv[pl.ds(i * 16, 16)] = (inv[pl.ds(i * 16, 16)]
                                              + ctmp[pl.ds(i * 16, 16)])
                    return 0
                lax.fori_loop(0, RPT // 16, _radd, 0)

            def _rinv(i, _):
                v = inv[pl.ds(i * 16, 16)]
                inv[pl.ds(i * 16, 16)] = 1.0 / jnp.maximum(v, 1.0)
                return 0
            lax.fori_loop(0, RPT // 16, _rinv, 0)

            @pl.when(cid == 0)
            def _winv():
                pltpu.sync_copy(inv.at[pl.ds(0, RPT)],
                                inv_out.at[g, pl.ds(r0, RPT)])
        else:
            pltpu.sync_copy(inv_in.at[g, pl.ds(r0, RPT)],
                            inv.at[pl.ds(0, RPT)])
            plsc.subcore_barrier()

        for ch in range(RPT // EPC):
            start = r0 + ch * EPC
            pltpu.sync_copy(acc.at[pl.ds(start, EPC), :], rows_a)
            if has_extra:
                @pl.when(start + EPC <= N)
                def _full_in():
                    pltpu.sync_copy(
                        extra.at[g, pl.ds(start, EPC), pl.ds(col0, CB)],
                        rows_b)

                @pl.when(jnp.logical_and(start < N, start + EPC > N))
                def _part_in():
                    pltpu.sync_copy(
                        extra.at[g, pl.ds(start, NTAIL), pl.ds(col0, CB)],
                        rows_b.at[pl.ds(0, NTAIL)])

            def _scale(r, _):
                vv = inv[pl.ds(ch * EPC + r, 16)]
                svec = jnp.full((16,), vv[0], jnp.float32)
                for c in range(CB // 16):
                    val = rows_a[r, pl.ds(c * 16, 16)] * svec
                    if has_extra:
                        val = val + rows_b[r, pl.ds(c * 16, 16)]
                    rows_a[r, pl.ds(c * 16, 16)] = val
                return 0
            lax.fori_loop(0, EPC, _scale, 0)

            if compute_counts:
                dst_full = outs[g].at[g, pl.ds(start, EPC), pl.ds(col0, CB)]
                dst_part = outs[g].at[g, pl.ds(start, NTAIL), pl.ds(col0, CB)]
            else:
                dst_full = outs[g].at[pl.ds(start, EPC), pl.ds(col0, CB)]
                dst_part = outs[g].at[pl.ds(start, NTAIL), pl.ds(col0, CB)]

            @pl.when(start + EPC <= N)
            def _full_out():
                pltpu.sync_copy(rows_a, dst_full)

            @pl.when(jnp.logical_and(start < N, start + EPC > N))
            def _part_out():
                pltpu.sync_copy(rows_a.at[pl.ds(0, NTAIL)], dst_part)


def _make_seg_mean(compute_counts):
    mesh = plsc.VectorSubcoreMesh(core_axis_name="c", subcore_axis_name="s")
    scratch = [
        pltpu.VMEM_SHARED((NP, CB), jnp.float32),
        pltpu.VMEM((K,), jnp.int32),
        pltpu.VMEM((K,), jnp.int32),
        pltpu.VMEM((K,), jnp.int32),
        pltpu.VMEM((K,), jnp.int32),
        pltpu.VMEM((K,), jnp.int32),
        pltpu.VMEM((K,), jnp.int32),
        pltpu.VMEM((K, CB), jnp.float32),
        pltpu.VMEM((K, CB), jnp.float32),
    ]
    if compute_counts:
        scratch += [
            pltpu.VMEM((NP,), jnp.float32),
            pltpu.VMEM((RPT,), jnp.float32),
        ]
    scratch += [
        pltpu.VMEM((RPT + 16,), jnp.float32),
        pltpu.SemaphoreType.DMA,
        pltpu.SemaphoreType.DMA,
        pltpu.SemaphoreType.DMA,
        pltpu.SemaphoreType.DMA,
        pltpu.SemaphoreType.DMA,
        pltpu.SemaphoreType.DMA,
    ]
    if compute_counts:
        out_type = [
            jax.ShapeDtypeStruct((2, N, D_IN), jnp.float32),
            jax.ShapeDtypeStruct((2, NP), jnp.float32),
            jax.ShapeDtypeStruct((2, NTILES, NP), jnp.float32),
        ]
    else:
        out_type = [
            jax.ShapeDtypeStruct((N, D_OUT), jnp.float32),
            jax.ShapeDtypeStruct((N, D_OUT), jnp.float32),
        ]

    @functools.partial(
        pl.kernel,
        out_type=out_type,
        mesh=mesh,
        scratch_types=scratch,
        compiler_params=pltpu.CompilerParams(needs_layout_passes=False),
    )
    def k(*args):
        _seg_mean_kernel_body(compute_counts, args)

    return k


_seg_mean_cache = {}


def _get_seg_mean(compute_counts):
    if compute_counts not in _seg_mean_cache:
        _seg_mean_cache[compute_counts] = _make_seg_mean(compute_counts)
    return _seg_mean_cache[compute_counts]


BM = 1000


def _tc_body(mean_r, x_r, w1l_r, w1r_r, b1l_r, w2l_r, w2r_r, b2l_r,
             hl_r, hr_r):
    dn = (((1,), (1,)), ((), ()))
    bf = jnp.bfloat16
    m = mean_r[0].astype(bf)
    xx = x_r[0].astype(bf)
    h1 = lax.dot_general(m, w1l_r[...].astype(bf), dn,
                         preferred_element_type=jnp.float32)
    h1 = h1 + lax.dot_general(xx, w1r_r[...].astype(bf), dn,
                              preferred_element_type=jnp.float32)
    h1 = jnp.maximum(h1 + b1l_r[...], 0.0).astype(bf)
    hl_r[0] = lax.dot_general(h1, w2l_r[...].astype(bf), dn,
                              preferred_element_type=jnp.float32)
    hr_r[0] = (lax.dot_general(h1, w2r_r[...].astype(bf), dn,
                               preferred_element_type=jnp.float32)
               + b2l_r[...])


def _tc_fused(mean1, x, W1l, b1l, W1r, W2l, b2l, W2r):
    grid = (2, N // BM)
    row_spec = pl.BlockSpec((1, BM, D_IN), lambda g, i: (g, i, 0))
    w_spec = pl.BlockSpec((D_HID, D_IN), lambda g, i: (0, 0))
    w2_spec = pl.BlockSpec((D_OUT, D_HID), lambda g, i: (0, 0))
    b1_spec = pl.BlockSpec((1, D_HID), lambda g, i: (0, 0))
    b2_spec = pl.BlockSpec((1, D_OUT), lambda g, i: (0, 0))
    out_spec = pl.BlockSpec((1, BM, D_OUT), lambda g, i: (g, i, 0))
    return pl.pallas_call(
        _tc_body,
        grid=grid,
        in_specs=[row_spec, row_spec, w_spec, w_spec, b1_spec,
                  w2_spec, w2_spec, b2_spec],
        out_specs=[out_spec, out_spec],
        out_shape=[jax.ShapeDtypeStruct((2, N, D_OUT), jnp.float32),
                   jax.ShapeDtypeStruct((2, N, D_OUT), jnp.float32)],
        compiler_params=pltpu.CompilerParams(
            dimension_semantics=("parallel", "parallel")),
    )(mean1, x, W1l, W1r, b1l.reshape(1, D_HID), W2l, W2r,
      b2l.reshape(1, D_OUT))


def kernel(x, edge_index, edge_index_min, W1l, b1l, W1r, W2l, b2l, W2r):
    src_i = edge_index[0].astype(jnp.int32)
    dst_i = edge_index[1].astype(jnp.int32)
    src_u = edge_index_min[0].astype(jnp.int32)
    dst_u = edge_index_min[1].astype(jnp.int32)

    x2 = x.reshape(2 * N, D_IN)
    mean1, invd, _ = _get_seg_mean(True)(x2, src_u, dst_u, src_i, dst_i)

    hl, hr = _tc_fused(mean1, x, W1l, b1l, W1r, W2l, b2l, W2r)

    out_u, out_i = _get_seg_mean(False)(hl.reshape(2 * N, D_OUT), src_u,
                                        dst_u, src_i, dst_i, invd, hr)
    return out_u, out_i


# speedup vs baseline: 7.2222x; 1.0192x over previous
import functools

import jax
import jax.numpy as jnp
from jax import lax
from jax.experimental import pallas as pl
from jax.experimental.pallas import tpu as pltpu
from jax.experimental.pallas import tpu_sc as plsc

N = 10000
NP = 10240
D_IN = 256
D_HID = 512
D_OUT = 256
E_I = 160000
E_U = 80000
CB = 128
K = 128
NTILES = 16
RPT = NP // NTILES
EPC = 128
NTAIL = N % EPC


def _seg_mean_kernel_body(compute_counts, args):
    if compute_counts:
        (xflat, e_min, e_idx, out, inv_out, cstage,
         acc, raw_a, flat_a, dst_a, raw_b, flat_b, dst_b,
         rows_a, rows_b, cnt, ctmp, inv,
         is_a, is_b, gs_a, gs_b, ss_a, ss_b) = args
        extra = None
        outs = (out, out)
    else:
        (xflat, e_min, e_idx, inv_in, extra, out_u, out_i,
         acc, raw_a, flat_a, dst_a, raw_b, flat_b, dst_b,
         rows_a, rows_b, inv,
         is_a, is_b, gs_a, gs_b, ss_a, ss_b) = args
        outs = (out_u, out_i)

    has_extra = extra is not None
    cid = lax.axis_index("c")
    tid = lax.axis_index("s")
    r0 = tid * RPT
    zeros16 = jnp.zeros((16,), jnp.float32)
    ones16 = jnp.ones((16,), jnp.float32)

    for g, (er, e_total) in enumerate(((e_min, E_U), (e_idx, E_I))):
        chunks_total = e_total // K
        base = chunks_total // NTILES
        nc = jnp.where(tid == NTILES - 1,
                       chunks_total - (NTILES - 1) * base, base)
        lo_c = tid * base

        off = g * N
        col0 = cid * CB

        def src_start(c, rb, sem):
            e0 = (lo_c + c) * K
            pltpu.async_copy(er.at[0, pl.ds(e0, K)], rb, sem)

        def dst_start(c, db, sem):
            e0 = (lo_c + c) * K
            pltpu.async_copy(er.at[1, pl.ds(e0, K)], db, sem)

        def idx_start(c, rb, db, sem):
            src_start(c, rb, sem)
            dst_start(c, db, sem)

        def idx_wait(c, rb, db, sem):
            e0 = (lo_c + c) * K
            pltpu.make_async_copy(er.at[0, pl.ds(e0, K)], rb, sem).wait()
            pltpu.make_async_copy(er.at[1, pl.ds(e0, K)], db, sem).wait()

        def transform(rb, fb, db):
            for k2 in range(K // 16):
                sv = rb[pl.ds(k2 * 16, 16)]
                fb[pl.ds(k2 * 16, 16)] = sv + off
                if compute_counts:
                    dv = db[pl.ds(k2 * 16, 16)]
                    plsc.addupdate_scatter(cnt, [dv], ones16)

        def gather_start(fb, rows, sem):
            pltpu.async_copy(xflat.at[fb, pl.ds(col0, CB)], rows, sem)

        def gather_wait(fb, rows, sem):
            pltpu.make_async_copy(xflat.at[fb, pl.ds(col0, CB)],
                                  rows, sem).wait()

        def scat_start(rows, db, sem):
            pltpu.async_copy(rows, acc.at[db], sem, add=True)

        def scat_wait(rows, db, sem):
            pltpu.make_async_copy(rows, acc.at[db], sem).wait()

        def _zrow(r, _):
            for c in range(CB // 16):
                rows_a[r, pl.ds(c * 16, 16)] = zeros16
            return 0
        lax.fori_loop(0, K, _zrow, 0)
        for z in range(RPT // K):
            pltpu.sync_copy(rows_a, acc.at[pl.ds(r0 + z * K, K), :])
        if compute_counts:
            def _zcnt(i, _):
                cnt[pl.ds(i * 16, 16)] = zeros16
                return 0
            lax.fori_loop(0, NP // 16, _zcnt, 0)
        plsc.subcore_barrier()

        idx_start(0, raw_a, dst_a, is_a)

        @pl.when(nc > 1)
        def _pro_b():
            idx_start(1, raw_b, dst_b, is_b)

        idx_wait(0, raw_a, dst_a, is_a)
        transform(raw_a, flat_a, dst_a)
        gather_start(flat_a, rows_a, gs_a)

        def _pair(p, _):
            c1 = 2 * p + 1
            c2 = c1 + 1
            c3 = c1 + 2
            has1 = c1 < nc

            @pl.when(c2 < nc)
            def _pre_a():
                src_start(c2, raw_a, is_a)

            @pl.when(has1)
            def _prep_b():
                idx_wait(c1, raw_b, dst_b, is_b)
                transform(raw_b, flat_b, dst_b)

            @pl.when(c3 < nc)
            def _pre_b():
                src_start(c3, raw_b, is_b)

            gather_wait(flat_a, rows_a, gs_a)
            scat_start(rows_a, dst_a, ss_a)

            @pl.when(has1)
            def _g_b():
                gather_start(flat_b, rows_b, gs_b)

            scat_wait(rows_a, dst_a, ss_a)

            @pl.when(c2 < nc)
            def _next_a():
                dst_start(c2, dst_a, is_a)
                idx_wait(c2, raw_a, dst_a, is_a)
                transform(raw_a, flat_a, dst_a)
                gather_start(flat_a, rows_a, gs_a)

            @pl.when(has1)
            def _fin_b():
                gather_wait(flat_b, rows_b, gs_b)
                scat_start(rows_b, dst_b, ss_b)
                scat_wait(rows_b, dst_b, ss_b)

            @pl.when(c3 < nc)
            def _idx_b():
                dst_start(c3, dst_b, is_b)
            return 0
        lax.fori_loop(0, (nc + 1) // 2, _pair, 0)

        if compute_counts:
            pltpu.sync_copy(cnt, cstage.at[g, tid])
            plsc.subcore_barrier()

            def _zinv(i, _):
                inv[pl.ds(i * 16, 16)] = zeros16
                return 0
            lax.fori_loop(0, RPT // 16, _zinv, 0)
            for k in range(NTILES):
                pltpu.sync_copy(cstage.at[g, k, pl.ds(r0, RPT)], ctmp)

                def _radd(i, _):
                    inv[pl.ds(i * 16, 16)] = (inv[pl.ds(i * 16, 16)]
                                              + ctmp[pl.ds(i * 16, 16)])
                    return 0
                lax.fori_loop(0, RPT // 16, _radd, 0)

            def _rinv(i, _):
                v = inv[pl.ds(i * 16, 16)]
                inv[pl.ds(i * 16, 16)] = 1.0 / jnp.maximum(v, 1.0)
                return 0
            lax.fori_loop(0, RPT // 16, _rinv, 0)

            @pl.when(cid == 0)
            def _winv():
                pltpu.sync_copy(inv.at[pl.ds(0, RPT)],
                                inv_out.at[g, pl.ds(r0, RPT)])
        else:
            pltpu.sync_copy(inv_in.at[g, pl.ds(r0, RPT)],
                            inv.at[pl.ds(0, RPT)])
            plsc.subcore_barrier()

        for ch in range(RPT // EPC):
            start = r0 + ch * EPC
            pltpu.sync_copy(acc.at[pl.ds(start, EPC), :], rows_a)
            if has_extra:
                @pl.when(start + EPC <= N)
                def _full_in():
                    pltpu.sync_copy(
                        extra.at[g, pl.ds(start, EPC), pl.ds(col0, CB)],
                        rows_b)

                @pl.when(jnp.logical_and(start < N, start + EPC > N))
                def _part_in():
                    pltpu.sync_copy(
                        extra.at[g, pl.ds(start, NTAIL), pl.ds(col0, CB)],
                        rows_b.at[pl.ds(0, NTAIL)])

            def _scale(r, _):
                vv = inv[pl.ds(ch * EPC + r, 16)]
                svec = jnp.full((16,), vv[0], jnp.float32)
                for c in range(CB // 16):
                    val = rows_a[r, pl.ds(c * 16, 16)] * svec
                    if has_extra:
                        val = val + rows_b[r, pl.ds(c * 16, 16)]
                    rows_a[r, pl.ds(c * 16, 16)] = val
                return 0
            lax.fori_loop(0, EPC, _scale, 0)

            if compute_counts:
                dst_full = outs[g].at[g, pl.ds(start, EPC), pl.ds(col0, CB)]
                dst_part = outs[g].at[g, pl.ds(start, NTAIL), pl.ds(col0, CB)]
            else:
                dst_full = outs[g].at[pl.ds(start, EPC), pl.ds(col0, CB)]
                dst_part = outs[g].at[pl.ds(start, NTAIL), pl.ds(col0, CB)]

            @pl.when(start + EPC <= N)
            def _full_out():
                pltpu.sync_copy(rows_a, dst_full)

            @pl.when(jnp.logical_and(start < N, start + EPC > N))
            def _part_out():
                pltpu.sync_copy(rows_a.at[pl.ds(0, NTAIL)], dst_part)


def _make_seg_mean(compute_counts):
    mesh = plsc.VectorSubcoreMesh(core_axis_name="c", subcore_axis_name="s")
    scratch = [
        pltpu.VMEM_SHARED((NP, CB), jnp.float32),
        pltpu.VMEM((K,), jnp.int32),
        pltpu.VMEM((K,), jnp.int32),
        pltpu.VMEM((K,), jnp.int32),
        pltpu.VMEM((K,), jnp.int32),
        pltpu.VMEM((K,), jnp.int32),
        pltpu.VMEM((K,), jnp.int32),
        pltpu.VMEM((K, CB), jnp.float32),
        pltpu.VMEM((K, CB), jnp.float32),
    ]
    if compute_counts:
        scratch += [
            pltpu.VMEM((NP,), jnp.float32),
            pltpu.VMEM((RPT,), jnp.float32),
        ]
    scratch += [
        pltpu.VMEM((RPT + 16,), jnp.float32),
        pltpu.SemaphoreType.DMA,
        pltpu.SemaphoreType.DMA,
        pltpu.SemaphoreType.DMA,
        pltpu.SemaphoreType.DMA,
        pltpu.SemaphoreType.DMA,
        pltpu.SemaphoreType.DMA,
    ]
    if compute_counts:
        out_type = [
            jax.ShapeDtypeStruct((2, N, D_IN), jnp.float32),
            jax.ShapeDtypeStruct((2, NP), jnp.float32),
            jax.ShapeDtypeStruct((2, NTILES, NP), jnp.float32),
        ]
    else:
        out_type = [
            jax.ShapeDtypeStruct((N, D_OUT), jnp.float32),
            jax.ShapeDtypeStruct((N, D_OUT), jnp.float32),
        ]

    @functools.partial(
        pl.kernel,
        out_type=out_type,
        mesh=mesh,
        scratch_types=scratch,
        compiler_params=pltpu.CompilerParams(needs_layout_passes=False),
    )
    def k(*args):
        _seg_mean_kernel_body(compute_counts, args)

    return k


_seg_mean_cache = {}


def _get_seg_mean(compute_counts):
    if compute_counts not in _seg_mean_cache:
        _seg_mean_cache[compute_counts] = _make_seg_mean(compute_counts)
    return _seg_mean_cache[compute_counts]


BM = 1000


def _tc_body(mean_r, x_r, w1l_r, w1r_r, b1l_r, w2l_r, w2r_r, b2l_r,
             hl_r, hr_r):
    dn = (((1,), (1,)), ((), ()))
    bf = jnp.bfloat16
    m = mean_r[0].astype(bf)
    xx = x_r[0].astype(bf)
    h1 = lax.dot_general(m, w1l_r[...].astype(bf), dn,
                         preferred_element_type=jnp.float32)
    h1 = h1 + lax.dot_general(xx, w1r_r[...].astype(bf), dn,
                              preferred_element_type=jnp.float32)
    h1 = jnp.maximum(h1 + b1l_r[...], 0.0).astype(bf)
    hl_r[0] = lax.dot_general(h1, w2l_r[...].astype(bf), dn,
                              preferred_element_type=jnp.float32)
    hr_r[0] = (lax.dot_general(h1, w2r_r[...].astype(bf), dn,
                               preferred_element_type=jnp.float32)
               + b2l_r[...])


def _tc_fused(mean1, x, W1l, b1l, W1r, W2l, b2l, W2r):
    grid = (2, N // BM)
    row_spec = pl.BlockSpec((1, BM, D_IN), lambda g, i: (g, i, 0))
    w_spec = pl.BlockSpec((D_HID, D_IN), lambda g, i: (0, 0))
    w2_spec = pl.BlockSpec((D_OUT, D_HID), lambda g, i: (0, 0))
    b1_spec = pl.BlockSpec((1, D_HID), lambda g, i: (0, 0))
    b2_spec = pl.BlockSpec((1, D_OUT), lambda g, i: (0, 0))
    out_spec = pl.BlockSpec((1, BM, D_OUT), lambda g, i: (g, i, 0))
    return pl.pallas_call(
        _tc_body,
        grid=grid,
        in_specs=[row_spec, row_spec, w_spec, w_spec, b1_spec,
                  w2_spec, w2_spec, b2_spec],
        out_specs=[out_spec, out_spec],
        out_shape=[jax.ShapeDtypeStruct((2, N, D_OUT), jnp.float32),
                   jax.ShapeDtypeStruct((2, N, D_OUT), jnp.float32)],
        compiler_params=pltpu.CompilerParams(
            dimension_semantics=("parallel", "parallel")),
    )(mean1, x, W1l, W1r, b1l.reshape(1, D_HID), W2l, W2r,
      b2l.reshape(1, D_OUT))


def kernel(x, edge_index, edge_index_min, W1l, b1l, W1r, W2l, b2l, W2r):
    e_idx = edge_index.astype(jnp.int32)
    e_min = edge_index_min.astype(jnp.int32)

    x2 = x.reshape(2 * N, D_IN)
    mean1, invd, _ = _get_seg_mean(True)(x2, e_min, e_idx)

    hl, hr = _tc_fused(mean1, x, W1l, b1l, W1r, W2l, b2l, W2r)

    out_u, out_i = _get_seg_mean(False)(hl.reshape(2 * N, D_OUT),
                                        e_min, e_idx, invd, hr)
    return out_u, out_i


# speedup vs baseline: 7.5081x; 1.0396x over previous
import functools

import jax
import jax.numpy as jnp
from jax import lax
from jax.experimental import pallas as pl
from jax.experimental.pallas import tpu as pltpu
from jax.experimental.pallas import tpu_sc as plsc

N = 10000
NP = 10240
D_IN = 256
D_HID = 512
D_OUT = 256
E_I = 160000
E_U = 80000
CB = 128
K = 128
NTILES = 16
RPT = NP // NTILES
EPC = 128
NTAIL = N % EPC


def _seg_mean_kernel_body(compute_counts, off, e_total, args):
    if compute_counts:
        (xflat, er, out, inv_out, cstage,
         acc, raw_a, flat_a, dst_a, raw_b, flat_b, dst_b,
         rows_a, rows_b, cnt, ctmp, inv,
         is_a, is_b, gs_a, gs_b, ss_a, ss_b) = args
        extra = None
    else:
        (xflat, er, inv_in, extra, out,
         acc, raw_a, flat_a, dst_a, raw_b, flat_b, dst_b,
         rows_a, rows_b, inv,
         is_a, is_b, gs_a, gs_b, ss_a, ss_b) = args

    has_extra = extra is not None
    cid = lax.axis_index("c")
    tid = lax.axis_index("s")
    r0 = tid * RPT
    zeros16 = jnp.zeros((16,), jnp.float32)
    ones16 = jnp.ones((16,), jnp.float32)

    chunks_total = e_total // K
    base = chunks_total // NTILES
    nc = jnp.where(tid == NTILES - 1,
                   chunks_total - (NTILES - 1) * base, base)
    lo_c = tid * base
    col0 = cid * CB

    def src_start(c, rb, sem):
        e0 = (lo_c + c) * K
        pltpu.async_copy(er.at[0, pl.ds(e0, K)], rb, sem)

    def dst_start(c, db, sem):
        e0 = (lo_c + c) * K
        pltpu.async_copy(er.at[1, pl.ds(e0, K)], db, sem)

    def idx_start(c, rb, db, sem):
        src_start(c, rb, sem)
        dst_start(c, db, sem)

    def idx_wait(c, rb, db, sem):
        e0 = (lo_c + c) * K
        pltpu.make_async_copy(er.at[0, pl.ds(e0, K)], rb, sem).wait()
        pltpu.make_async_copy(er.at[1, pl.ds(e0, K)], db, sem).wait()

    def transform(rb, fb, db):
        for k2 in range(K // 16):
            sv = rb[pl.ds(k2 * 16, 16)]
            fb[pl.ds(k2 * 16, 16)] = sv + off
            if compute_counts:
                dv = db[pl.ds(k2 * 16, 16)]
                plsc.addupdate_scatter(cnt, [dv], ones16)

    def gather_start(fb, rows, sem):
        pltpu.async_copy(xflat.at[fb, pl.ds(col0, CB)], rows, sem)

    def gather_wait(fb, rows, sem):
        pltpu.make_async_copy(xflat.at[fb, pl.ds(col0, CB)],
                              rows, sem).wait()

    def scat_start(rows, db, sem):
        pltpu.async_copy(rows, acc.at[db], sem, add=True)

    def scat_wait(rows, db, sem):
        pltpu.make_async_copy(rows, acc.at[db], sem).wait()

    def _zrow(r, _):
        for c in range(CB // 16):
            rows_a[r, pl.ds(c * 16, 16)] = zeros16
        return 0
    lax.fori_loop(0, K, _zrow, 0)
    for z in range(RPT // K):
        pltpu.sync_copy(rows_a, acc.at[pl.ds(r0 + z * K, K), :])
    if compute_counts:
        def _zcnt(i, _):
            cnt[pl.ds(i * 16, 16)] = zeros16
            return 0
        lax.fori_loop(0, NP // 16, _zcnt, 0)
    plsc.subcore_barrier()

    idx_start(0, raw_a, dst_a, is_a)

    @pl.when(nc > 1)
    def _pro_b():
        idx_start(1, raw_b, dst_b, is_b)

    idx_wait(0, raw_a, dst_a, is_a)
    transform(raw_a, flat_a, dst_a)
    gather_start(flat_a, rows_a, gs_a)

    def _pair(p, _):
        c1 = 2 * p + 1
        c2 = c1 + 1
        c3 = c1 + 2
        has1 = c1 < nc

        @pl.when(c2 < nc)
        def _pre_a():
            src_start(c2, raw_a, is_a)

        @pl.when(has1)
        def _prep_b():
            idx_wait(c1, raw_b, dst_b, is_b)
            transform(raw_b, flat_b, dst_b)

        @pl.when(c3 < nc)
        def _pre_b():
            src_start(c3, raw_b, is_b)

        gather_wait(flat_a, rows_a, gs_a)
        scat_start(rows_a, dst_a, ss_a)

        @pl.when(has1)
        def _g_b():
            gather_start(flat_b, rows_b, gs_b)

        scat_wait(rows_a, dst_a, ss_a)

        @pl.when(c2 < nc)
        def _next_a():
            dst_start(c2, dst_a, is_a)
            idx_wait(c2, raw_a, dst_a, is_a)
            transform(raw_a, flat_a, dst_a)
            gather_start(flat_a, rows_a, gs_a)

        @pl.when(has1)
        def _fin_b():
            gather_wait(flat_b, rows_b, gs_b)
            scat_start(rows_b, dst_b, ss_b)
            scat_wait(rows_b, dst_b, ss_b)

        @pl.when(c3 < nc)
        def _idx_b():
            dst_start(c3, dst_b, is_b)
        return 0
    lax.fori_loop(0, (nc + 1) // 2, _pair, 0)

    if compute_counts:
        pltpu.sync_copy(cnt, cstage.at[tid])
        plsc.subcore_barrier()

        def _zinv(i, _):
            inv[pl.ds(i * 16, 16)] = zeros16
            return 0
        lax.fori_loop(0, RPT // 16, _zinv, 0)
        for k in range(NTILES):
            pltpu.sync_copy(cstage.at[k, pl.ds(r0, RPT)], ctmp)

            def _radd(i, _):
                inv[pl.ds(i * 16, 16)] = (inv[pl.ds(i * 16, 16)]
                                          + ctmp[pl.ds(i * 16, 16)])
                return 0
            lax.fori_loop(0, RPT // 16, _radd, 0)

        def _rinv(i, _):
            v = inv[pl.ds(i * 16, 16)]
            inv[pl.ds(i * 16, 16)] = 1.0 / jnp.maximum(v, 1.0)
            return 0
        lax.fori_loop(0, RPT // 16, _rinv, 0)

        @pl.when(cid == 0)
        def _winv():
            pltpu.sync_copy(inv.at[pl.ds(0, RPT)],
                            inv_out.at[pl.ds(r0, RPT)])
    else:
        pltpu.sync_copy(inv_in.at[pl.ds(r0, RPT)], inv.at[pl.ds(0, RPT)])
        plsc.subcore_barrier()

    for ch in range(RPT // EPC):
        start = r0 + ch * EPC
        pltpu.sync_copy(acc.at[pl.ds(start, EPC), :], rows_a)
        if has_extra:
            @pl.when(start + EPC <= N)
            def _full_in():
                pltpu.sync_copy(
                    extra.at[pl.ds(start, EPC), pl.ds(col0, CB)], rows_b)

            @pl.when(jnp.logical_and(start < N, start + EPC > N))
            def _part_in():
                pltpu.sync_copy(
                    extra.at[pl.ds(start, NTAIL), pl.ds(col0, CB)],
                    rows_b.at[pl.ds(0, NTAIL)])

        def _scale(r, _):
            vv = inv[pl.ds(ch * EPC + r, 16)]
            svec = jnp.full((16,), vv[0], jnp.float32)
            for c in range(CB // 16):
                val = rows_a[r, pl.ds(c * 16, 16)] * svec
                if has_extra:
                    val = val + rows_b[r, pl.ds(c * 16, 16)]
                rows_a[r, pl.ds(c * 16, 16)] = val
            return 0
        lax.fori_loop(0, EPC, _scale, 0)

        @pl.when(start + EPC <= N)
        def _full_out():
            pltpu.sync_copy(rows_a,
                            out.at[pl.ds(start, EPC), pl.ds(col0, CB)])

        @pl.when(jnp.logical_and(start < N, start + EPC > N))
        def _part_out():
            pltpu.sync_copy(rows_a.at[pl.ds(0, NTAIL)],
                            out.at[pl.ds(start, NTAIL), pl.ds(col0, CB)])


def _make_seg_mean(compute_counts, off, e_total, d_feat):
    mesh = plsc.VectorSubcoreMesh(core_axis_name="c", subcore_axis_name="s")
    scratch = [
        pltpu.VMEM_SHARED((NP, CB), jnp.float32),
        pltpu.VMEM((K,), jnp.int32),
        pltpu.VMEM((K,), jnp.int32),
        pltpu.VMEM((K,), jnp.int32),
        pltpu.VMEM((K,), jnp.int32),
        pltpu.VMEM((K,), jnp.int32),
        pltpu.VMEM((K,), jnp.int32),
        pltpu.VMEM((K, CB), jnp.float32),
        pltpu.VMEM((K, CB), jnp.float32),
    ]
    if compute_counts:
        scratch += [
            pltpu.VMEM((NP,), jnp.float32),
            pltpu.VMEM((RPT,), jnp.float32),
        ]
    scratch += [
        pltpu.VMEM((RPT + 16,), jnp.float32),
        pltpu.SemaphoreType.DMA,
        pltpu.SemaphoreType.DMA,
        pltpu.SemaphoreType.DMA,
        pltpu.SemaphoreType.DMA,
        pltpu.SemaphoreType.DMA,
        pltpu.SemaphoreType.DMA,
    ]
    if compute_counts:
        out_type = [
            jax.ShapeDtypeStruct((N, d_feat), jnp.float32),
            jax.ShapeDtypeStruct((NP,), jnp.float32),
            jax.ShapeDtypeStruct((NTILES, NP), jnp.float32),
        ]
    else:
        out_type = jax.ShapeDtypeStruct((N, d_feat), jnp.float32)

    @functools.partial(
        pl.kernel,
        out_type=out_type,
        mesh=mesh,
        scratch_types=scratch,
        compiler_params=pltpu.CompilerParams(needs_layout_passes=False),
    )
    def k(*args):
        _seg_mean_kernel_body(compute_counts, off, e_total, args)

    return k


_seg_mean_cache = {}


def _get_seg_mean(compute_counts, off, e_total):
    key = (compute_counts, off, e_total)
    if key not in _seg_mean_cache:
        _seg_mean_cache[key] = _make_seg_mean(compute_counts, off, e_total,
                                              D_IN)
    return _seg_mean_cache[key]


BM = 1000


def _tc_body(mean_r, x_r, w1l_r, w1r_r, b1l_r, w2l_r, w2r_r, b2l_r,
             hl_r, hr_r):
    dn = (((1,), (1,)), ((), ()))
    bf = jnp.bfloat16
    m = mean_r[...].astype(bf)
    xx = x_r[0].astype(bf)
    h1 = lax.dot_general(m, w1l_r[...].astype(bf), dn,
                         preferred_element_type=jnp.float32)
    h1 = h1 + lax.dot_general(xx, w1r_r[...].astype(bf), dn,
                              preferred_element_type=jnp.float32)
    h1 = jnp.maximum(h1 + b1l_r[...], 0.0).astype(bf)
    hl_r[...] = lax.dot_general(h1, w2l_r[...].astype(bf), dn,
                                preferred_element_type=jnp.float32)
    hr_r[...] = (lax.dot_general(h1, w2r_r[...].astype(bf), dn,
                                 preferred_element_type=jnp.float32)
                 + b2l_r[...])


def _tc_fused(g, mean1, x, W1l, b1l, W1r, W2l, b2l, W2r):
    grid = (N // BM,)
    mean_spec = pl.BlockSpec((BM, D_IN), lambda i: (i, 0))
    x_spec = pl.BlockSpec((1, BM, D_IN), lambda i: (g, i, 0))
    w_spec = pl.BlockSpec((D_HID, D_IN), lambda i: (0, 0))
    w2_spec = pl.BlockSpec((D_OUT, D_HID), lambda i: (0, 0))
    b1_spec = pl.BlockSpec((1, D_HID), lambda i: (0, 0))
    b2_spec = pl.BlockSpec((1, D_OUT), lambda i: (0, 0))
    out_spec = pl.BlockSpec((BM, D_OUT), lambda i: (i, 0))
    return pl.pallas_call(
        _tc_body,
        grid=grid,
        in_specs=[mean_spec, x_spec, w_spec, w_spec, b1_spec,
                  w2_spec, w2_spec, b2_spec],
        out_specs=[out_spec, out_spec],
        out_shape=[jax.ShapeDtypeStruct((N, D_OUT), jnp.float32),
                   jax.ShapeDtypeStruct((N, D_OUT), jnp.float32)],
        compiler_params=pltpu.CompilerParams(
            dimension_semantics=("parallel",)),
    )(mean1, x, W1l, W1r, b1l.reshape(1, D_HID), W2l, W2r,
      b2l.reshape(1, D_OUT))


def kernel(x, edge_index, edge_index_min, W1l, b1l, W1r, W2l, b2l, W2r):
    e_idx = edge_index.astype(jnp.int32)
    e_min = edge_index_min.astype(jnp.int32)

    x2 = x.reshape(2 * N, D_IN)
    outs = []
    for g, er, e_tot in ((0, e_min, E_U), (1, e_idx, E_I)):
        mean_g, inv_g, _ = _get_seg_mean(True, g * N, e_tot)(x2, er)
        hl_g, hr_g = _tc_fused(g, mean_g, x, W1l, b1l, W1r, W2l, b2l, W2r)
        out_g = _get_seg_mean(False, 0, e_tot)(hl_g, er, inv_g, hr_g)
        outs.append(out_g)
    return outs[0], outs[1]
